# baseline (speedup 1.0000x reference)
import functools

import jax
import jax.numpy as jnp
from jax import lax
from jax.experimental import pallas as pl
from jax.experimental.pallas import tpu as pltpu
from jax.experimental.pallas import tpu_sc as plsc

N = 10000
N_PAD = 10240
E = 320000
CHUNK = 128
NT = 32
CPT = 80
SPARE_ROWS = 128
E2D_ROWS = NT * CPT + SPARE_ROWS
E_PAD = E2D_ROWS * CHUNK
D_IN = 128
D_H = 128
D_OUT = 64
ROWS_PT = N_PAD // 16

_mesh = lambda: plsc.VectorSubcoreMesh(
    core_axis_name="c", subcore_axis_name="s", num_cores=2, num_subcores=16
)


def _deg_kernel(dst_hbm, out_hbm, idx_v, ones_v, deg_sh):
    c = lax.axis_index("c")
    s = lax.axis_index("s")
    wid = c * 16 + s

    pltpu.sync_copy(dst_hbm.at[pl.ds(wid * CPT, CPT)], idx_v)

    @pl.loop(0, CHUNK // 16)
    def _(i):
        ones_v[pl.ds(i * 16, 16)] = jnp.zeros((16,), jnp.float32)

    @pl.loop(0, ROWS_PT // CHUNK)
    def _(i):
        pltpu.sync_copy(ones_v, deg_sh.at[pl.ds(s * ROWS_PT + i * CHUNK, CHUNK)])

    @pl.loop(0, CHUNK // 16)
    def _(i):
        ones_v[pl.ds(i * 16, 16)] = jnp.full((16,), 1.0, jnp.float32)

    plsc.subcore_barrier()

    @pl.loop(0, CPT)
    def _(j):
        pltpu.sync_copy(ones_v, deg_sh.at[idx_v.at[j]], add=True)

    plsc.subcore_barrier()
    pltpu.sync_copy(
        deg_sh.at[pl.ds(s * ROWS_PT, ROWS_PT)],
        out_hbm.at[c, pl.ds(s * ROWS_PT, ROWS_PT)],
    )


def _make_deg():
    return pl.kernel(
        _deg_kernel,
        out_type=jax.ShapeDtypeStruct((2, N_PAD), jnp.float32),
        mesh=_mesh(),
        scratch_types=[
            pltpu.VMEM((CPT, CHUNK), jnp.int32),
            pltpu.VMEM((CHUNK,), jnp.float32),
            pltpu.VMEM_SHARED((N_PAD,), jnp.float32),
        ],
    )


DW = 64


def _scatter_kernel(D, cpt0, yw_hbm, src_hbm, dst_hbm, out_hbm, src_v, dst_v, msg_v, acc_sh):
    c = lax.axis_index("c")
    s = lax.axis_index("s")
    cpt1 = 2 * CPT - cpt0
    cpt_max = max(cpt0, cpt1)
    cnt = jnp.where(c == 0, cpt0, cpt1)
    base = jnp.where(c == 0, s * cpt0, 16 * cpt0 + s * cpt1)

    pltpu.sync_copy(src_hbm.at[pl.ds(base, cpt_max)], src_v)
    pltpu.sync_copy(dst_hbm.at[pl.ds(base, cpt_max)], dst_v)

    @pl.loop(0, CHUNK)
    def _(r):
        @pl.loop(0, D // 16)
        def _(q):
            msg_v[r, pl.ds(q * 16, 16)] = jnp.zeros((16,), jnp.float32)

    @pl.loop(0, ROWS_PT // CHUNK)
    def _(i):
        pltpu.sync_copy(msg_v, acc_sh.at[pl.ds(s * ROWS_PT + i * CHUNK, CHUNK)])

    plsc.subcore_barrier()

    @pl.loop(0, cpt_max)
    def _(j):
        @pl.when(j < cnt)
        def _():
            pltpu.sync_copy(yw_hbm.at[src_v.at[j]], msg_v)
            pltpu.sync_copy(msg_v, acc_sh.at[dst_v.at[j]], add=True)

    plsc.subcore_barrier()
    pltpu.sync_copy(
        acc_sh.at[pl.ds(s * ROWS_PT, ROWS_PT)],
        out_hbm.at[c, pl.ds(s * ROWS_PT, ROWS_PT)],
    )


def _make_scatter(D, cpt0):
    cpt_max = max(cpt0, 2 * CPT - cpt0)
    return pl.kernel(
        functools.partial(_scatter_kernel, D, cpt0),
        out_type=jax.ShapeDtypeStruct((2, N_PAD, D), jnp.float32),
        mesh=_mesh(),
        scratch_types=[
            pltpu.VMEM((cpt_max, CHUNK), jnp.int32),
            pltpu.VMEM((cpt_max, CHUNK), jnp.int32),
            pltpu.VMEM((CHUNK, D), jnp.float32),
            pltpu.VMEM_SHARED((N_PAD, D), jnp.float32),
        ],
    )


_BM = 2048


def _dinv_of(dp_ref):
    deg = dp_ref[0, :] + dp_ref[1, :] + 1.0
    return lax.rsqrt(deg)


def _k1_body(x_ref, w_ref, dp_ref, o_ref):
    dinv = _dinv_of(dp_ref)
    xw = jnp.dot(x_ref[...], w_ref[...], preferred_element_type=jnp.float32)
    o_ref[...] = xw * dinv[:, None]


def _k2_body(sp_ref, yw_ref, dp_ref, w_ref, b_ref, o_ref):
    dinv = _dinv_of(dp_ref)
    stot = sp_ref[0] + sp_ref[1] + yw_ref[...]
    h = jnp.maximum(stot * dinv[:, None] + b_ref[...], 0.0)
    o_ref[...] = jnp.dot(h, w_ref[...], preferred_element_type=jnp.float32) * dinv[:, None]


def _k3_body(sp_ref, yw_ref, dp_ref, b_ref, o_ref):
    dinv = _dinv_of(dp_ref)
    stot = sp_ref[0] + sp_ref[1] + yw_ref[...]
    o_ref[...] = (stot * dinv[:, None])[:, :D_OUT] + b_ref[...]


def _row_spec(d):
    return pl.BlockSpec((_BM, d), lambda i: (i, 0))


def _parts_spec(d):
    return pl.BlockSpec((2, _BM, d), lambda i: (0, i, 0))


_DP_SPEC = pl.BlockSpec((2, _BM), lambda i: (0, i))
_GRID = (N_PAD // _BM,)


def _k1(x_pad, W1, deg_parts):
    return pl.pallas_call(
        _k1_body,
        grid=_GRID,
        in_specs=[
            _row_spec(D_IN),
            pl.BlockSpec((D_IN, D_H), lambda i: (0, 0)),
            _DP_SPEC,
        ],
        out_specs=_row_spec(D_H),
        out_shape=jax.ShapeDtypeStruct((N_PAD, D_H), jnp.float32),
    )(x_pad, W1, deg_parts)


def _k2(s1_parts, yw1, deg_parts, W2, b1_2d):
    return pl.pallas_call(
        _k2_body,
        grid=_GRID,
        in_specs=[
            _parts_spec(D_H),
            _row_spec(D_H),
            _DP_SPEC,
            pl.BlockSpec((D_H, D_H), lambda i: (0, 0)),
            pl.BlockSpec((1, D_H), lambda i: (0, 0)),
        ],
        out_specs=_row_spec(D_H),
        out_shape=jax.ShapeDtypeStruct((N_PAD, D_H), jnp.float32),
    )(s1_parts, yw1, deg_parts, W2, b1_2d)


def _k3(s2_parts, yw2, deg_parts, b2_2d):
    return pl.pallas_call(
        _k3_body,
        grid=_GRID,
        in_specs=[
            _parts_spec(D_H),
            _row_spec(D_H),
            _DP_SPEC,
            pl.BlockSpec((1, D_OUT), lambda i: (0, 0)),
        ],
        out_specs=_row_spec(D_OUT),
        out_shape=jax.ShapeDtypeStruct((N_PAD, D_OUT), jnp.float32),
    )(s2_parts, yw2, deg_parts, b2_2d)


@jax.jit
def _run(x, edge_index, W1, b1, W2, b2):
    x_pad = jnp.pad(x, ((0, N_PAD - N), (0, 0)))
    ei = edge_index.astype(jnp.int32)
    pad_idx = jnp.full((E_PAD - E,), N_PAD - 1, jnp.int32)
    src2d = jnp.concatenate([ei[0], pad_idx]).reshape(E2D_ROWS, CHUNK)
    dst2d = jnp.concatenate([ei[1], pad_idx]).reshape(E2D_ROWS, CHUNK)

    deg_parts = _make_deg()(dst2d)
    yw1 = _k1(x_pad, W1, deg_parts)
    s1_parts = _make_scatter(D_H, 80)(yw1, src2d, dst2d)
    W2p = jnp.pad(W2, ((0, 0), (0, D_H - D_OUT)))
    yw2 = _k2(s1_parts, yw1, deg_parts, W2p, b1.reshape(1, D_H))
    s2_parts = _make_scatter(D_H, 120)(yw2, src2d, dst2d)
    out = _k3(s2_parts, yw2, deg_parts, b2.reshape(1, D_OUT))
    return out[:N]


def kernel(x, edge_index, W1, b1, W2, b2):
    return _run(x, edge_index, W1, b1, W2, b2)


# speedup vs baseline: 1.0857x; 1.0857x over previous
import functools

import jax
import jax.numpy as jnp
from jax import lax
from jax.experimental import pallas as pl
from jax.experimental.pallas import tpu as pltpu
from jax.experimental.pallas import tpu_sc as plsc

N = 10000
N_PAD = 10240
E = 320000
CHUNK = 128
NT = 32
CPT = 80
SPARE_ROWS = 128
E2D_ROWS = NT * CPT + SPARE_ROWS
E_PAD = E2D_ROWS * CHUNK
D_IN = 128
D_H = 128
D_OUT = 64
ROWS_PT = N_PAD // 16

_mesh = lambda: plsc.VectorSubcoreMesh(
    core_axis_name="c", subcore_axis_name="s", num_cores=2, num_subcores=16
)


def _deg_kernel(dst_hbm, out_hbm, idx_v, ones_v, deg_sh):
    c = lax.axis_index("c")
    s = lax.axis_index("s")
    wid = c * 16 + s

    pltpu.sync_copy(dst_hbm.at[pl.ds(wid * CPT, CPT)], idx_v)

    @pl.loop(0, CHUNK // 16)
    def _(i):
        ones_v[pl.ds(i * 16, 16)] = jnp.zeros((16,), jnp.float32)

    @pl.loop(0, ROWS_PT // CHUNK)
    def _(i):
        pltpu.sync_copy(ones_v, deg_sh.at[pl.ds(s * ROWS_PT + i * CHUNK, CHUNK)])

    @pl.loop(0, CHUNK // 16)
    def _(i):
        ones_v[pl.ds(i * 16, 16)] = jnp.full((16,), 1.0, jnp.float32)

    plsc.subcore_barrier()

    @pl.loop(0, CPT)
    def _(j):
        pltpu.sync_copy(ones_v, deg_sh.at[idx_v.at[j]], add=True)

    plsc.subcore_barrier()
    pltpu.sync_copy(
        deg_sh.at[pl.ds(s * ROWS_PT, ROWS_PT)],
        out_hbm.at[c, pl.ds(s * ROWS_PT, ROWS_PT)],
    )


def _make_deg():
    return pl.kernel(
        _deg_kernel,
        out_type=jax.ShapeDtypeStruct((2, N_PAD), jnp.float32),
        mesh=_mesh(),
        scratch_types=[
            pltpu.VMEM((CPT, CHUNK), jnp.int32),
            pltpu.VMEM((CHUNK,), jnp.float32),
            pltpu.VMEM_SHARED((N_PAD,), jnp.float32),
        ],
    )


DW = 64


def _scatter_kernel(D, cpt0, yw_hbm, src_hbm, dst_hbm, out_hbm, src_v, dst_v, msg_v, acc_sh):
    c = lax.axis_index("c")
    s = lax.axis_index("s")
    cpt1 = 2 * CPT - cpt0
    cpt_max = max(cpt0, cpt1)
    cnt = jnp.where(c == 0, cpt0, cpt1)
    base = jnp.where(c == 0, s * cpt0, 16 * cpt0 + s * cpt1)

    pltpu.sync_copy(src_hbm.at[pl.ds(base, cpt_max)], src_v)
    pltpu.sync_copy(dst_hbm.at[pl.ds(base, cpt_max)], dst_v)

    @pl.loop(0, CHUNK)
    def _(r):
        @pl.loop(0, D // 16)
        def _(q):
            msg_v[r, pl.ds(q * 16, 16)] = jnp.zeros((16,), jnp.float32)

    @pl.loop(0, ROWS_PT // CHUNK)
    def _(i):
        pltpu.sync_copy(msg_v, acc_sh.at[pl.ds(s * ROWS_PT + i * CHUNK, CHUNK)])

    plsc.subcore_barrier()

    @pl.loop(0, cpt_max)
    def _(j):
        @pl.when(j < cnt)
        def _():
            pltpu.sync_copy(yw_hbm.at[src_v.at[j]], msg_v)
            pltpu.sync_copy(msg_v, acc_sh.at[dst_v.at[j]], add=True)

    plsc.subcore_barrier()
    pltpu.sync_copy(
        acc_sh.at[pl.ds(s * ROWS_PT, ROWS_PT)],
        out_hbm.at[c, pl.ds(s * ROWS_PT, ROWS_PT)],
    )


def _make_scatter(D, cpt0):
    cpt_max = max(cpt0, 2 * CPT - cpt0)
    return pl.kernel(
        functools.partial(_scatter_kernel, D, cpt0),
        out_type=jax.ShapeDtypeStruct((2, N_PAD, D), jnp.float32),
        mesh=_mesh(),
        scratch_types=[
            pltpu.VMEM((cpt_max, CHUNK), jnp.int32),
            pltpu.VMEM((cpt_max, CHUNK), jnp.int32),
            pltpu.VMEM((CHUNK, D), jnp.float32),
            pltpu.VMEM_SHARED((N_PAD, D), jnp.float32),
        ],
    )


_BM = 2048


def _dinv_of(dp_ref):
    deg = dp_ref[0, :] + dp_ref[1, :] + 1.0
    return lax.rsqrt(deg)


def _k1_body(x_ref, w_ref, dp_ref, o_ref):
    dinv = _dinv_of(dp_ref)
    xw = jnp.dot(x_ref[...], w_ref[...], preferred_element_type=jnp.float32)
    o_ref[...] = xw * dinv[:, None]


def _k2_body(sp_ref, yw_ref, dp_ref, w_ref, b_ref, o_ref):
    dinv = _dinv_of(dp_ref)
    stot = sp_ref[0] + sp_ref[1] + yw_ref[...]
    h = jnp.maximum(stot * dinv[:, None] + b_ref[...], 0.0)
    o_ref[...] = jnp.dot(h, w_ref[...], preferred_element_type=jnp.float32) * dinv[:, None]


def _k3_body(sp_ref, yw_ref, dp_ref, b_ref, o_ref):
    dinv = _dinv_of(dp_ref)
    stot = sp_ref[0] + sp_ref[1] + yw_ref[...]
    o_ref[...] = (stot * dinv[:, None])[:, :D_OUT] + b_ref[...]


def _row_spec(d):
    return pl.BlockSpec((_BM, d), lambda i: (i, 0))


def _parts_spec(d):
    return pl.BlockSpec((2, _BM, d), lambda i: (0, i, 0))


_DP_SPEC = pl.BlockSpec((2, _BM), lambda i: (0, i))
_GRID = (N_PAD // _BM,)


def _k1(x_pad, W1, deg_parts):
    return pl.pallas_call(
        _k1_body,
        grid=_GRID,
        in_specs=[
            _row_spec(D_IN),
            pl.BlockSpec((D_IN, D_H), lambda i: (0, 0)),
            _DP_SPEC,
        ],
        out_specs=_row_spec(D_H),
        out_shape=jax.ShapeDtypeStruct((N_PAD, D_H), jnp.float32),
    )(x_pad, W1, deg_parts)


def _k2(s1_parts, yw1, deg_parts, W2, b1_2d):
    return pl.pallas_call(
        _k2_body,
        grid=_GRID,
        in_specs=[
            _parts_spec(D_H),
            _row_spec(D_H),
            _DP_SPEC,
            pl.BlockSpec((D_H, D_H), lambda i: (0, 0)),
            pl.BlockSpec((1, D_H), lambda i: (0, 0)),
        ],
        out_specs=_row_spec(D_H),
        out_shape=jax.ShapeDtypeStruct((N_PAD, D_H), jnp.float32),
    )(s1_parts, yw1, deg_parts, W2, b1_2d)


def _k3(s2_parts, yw2, deg_parts, b2_2d):
    return pl.pallas_call(
        _k3_body,
        grid=_GRID,
        in_specs=[
            _parts_spec(D_H),
            _row_spec(D_H),
            _DP_SPEC,
            pl.BlockSpec((1, D_OUT), lambda i: (0, 0)),
        ],
        out_specs=_row_spec(D_OUT),
        out_shape=jax.ShapeDtypeStruct((N_PAD, D_OUT), jnp.float32),
    )(s2_parts, yw2, deg_parts, b2_2d)


@jax.jit
def _run(x, edge_index, W1, b1, W2, b2):
    x_pad = jnp.pad(x, ((0, N_PAD - N), (0, 0)))
    ei = edge_index.astype(jnp.int32)
    pad_idx = jnp.full((E_PAD - E,), N_PAD - 1, jnp.int32)
    src2d = jnp.concatenate([ei[0], pad_idx]).reshape(E2D_ROWS, CHUNK)
    dst2d = jnp.concatenate([ei[1], pad_idx]).reshape(E2D_ROWS, CHUNK)

    deg_parts = _make_deg()(dst2d)
    yw1 = _k1(x_pad, W1, deg_parts)
    s1_parts = _make_scatter(D_H, 120)(yw1, src2d, dst2d)
    W2p = jnp.pad(W2, ((0, 0), (0, D_H - D_OUT)))
    yw2 = _k2(s1_parts, yw1, deg_parts, W2p, b1.reshape(1, D_H))
    s2_parts = _make_scatter(D_H, 120)(yw2, src2d, dst2d)
    out = _k3(s2_parts, yw2, deg_parts, b2.reshape(1, D_OUT))
    return out[:N]


def kernel(x, edge_index, W1, b1, W2, b2):
    return _run(x, edge_index, W1, b1, W2, b2)


# speedup vs baseline: 1.1251x; 1.0363x over previous
import functools

import jax
import jax.numpy as jnp
from jax import lax
from jax.experimental import pallas as pl
from jax.experimental.pallas import tpu as pltpu
from jax.experimental.pallas import tpu_sc as plsc

N = 10000
N_PAD = 10240
E = 320000
CHUNK = 128
NT = 32
CPT = 80
SPARE_ROWS = 128
E2D_ROWS = NT * CPT + SPARE_ROWS
E_PAD = E2D_ROWS * CHUNK
D_IN = 128
D_H = 128
D_OUT = 64
ROWS_PT = N_PAD // 16

_mesh = lambda: plsc.VectorSubcoreMesh(
    core_axis_name="c", subcore_axis_name="s", num_cores=2, num_subcores=16
)


def _deg_kernel(dst_hbm, out_hbm, idx_v, ones_v, deg_sh):
    c = lax.axis_index("c")
    s = lax.axis_index("s")
    wid = c * 16 + s

    pltpu.sync_copy(dst_hbm.at[pl.ds(wid * CPT, CPT)], idx_v)

    @pl.loop(0, CHUNK // 16)
    def _(i):
        ones_v[pl.ds(i * 16, 16)] = jnp.zeros((16,), jnp.float32)

    @pl.loop(0, ROWS_PT // CHUNK)
    def _(i):
        pltpu.sync_copy(ones_v, deg_sh.at[pl.ds(s * ROWS_PT + i * CHUNK, CHUNK)])

    @pl.loop(0, CHUNK // 16)
    def _(i):
        ones_v[pl.ds(i * 16, 16)] = jnp.full((16,), 1.0, jnp.float32)

    plsc.subcore_barrier()

    @pl.loop(0, CPT)
    def _(j):
        pltpu.sync_copy(ones_v, deg_sh.at[idx_v.at[j]], add=True)

    plsc.subcore_barrier()
    pltpu.sync_copy(
        deg_sh.at[pl.ds(s * ROWS_PT, ROWS_PT)],
        out_hbm.at[c, pl.ds(s * ROWS_PT, ROWS_PT)],
    )


def _make_deg():
    return pl.kernel(
        _deg_kernel,
        out_type=jax.ShapeDtypeStruct((2, N_PAD), jnp.float32),
        mesh=_mesh(),
        scratch_types=[
            pltpu.VMEM((CPT, CHUNK), jnp.int32),
            pltpu.VMEM((CHUNK,), jnp.float32),
            pltpu.VMEM_SHARED((N_PAD,), jnp.float32),
        ],
    )


DW = 64


def _scatter_kernel(D, cpt0, yw_hbm, src_hbm, dst_hbm, out_hbm, src_v, dst_v, msg_v, acc_sh):
    c = lax.axis_index("c")
    s = lax.axis_index("s")
    cpt1 = 2 * CPT - cpt0
    cpt_max = max(cpt0, cpt1)
    cnt = jnp.where(c == 0, cpt0, cpt1)
    base = jnp.where(c == 0, s * cpt0, 16 * cpt0 + s * cpt1)

    pltpu.sync_copy(src_hbm.at[pl.ds(base, cpt_max)], src_v)
    pltpu.sync_copy(dst_hbm.at[pl.ds(base, cpt_max)], dst_v)

    @pl.loop(0, CHUNK)
    def _(r):
        @pl.loop(0, D // 16)
        def _(q):
            msg_v[r, pl.ds(q * 16, 16)] = jnp.zeros((16,), jnp.float32)

    @pl.loop(0, ROWS_PT // CHUNK)
    def _(i):
        pltpu.sync_copy(msg_v, acc_sh.at[pl.ds(s * ROWS_PT + i * CHUNK, CHUNK)])

    plsc.subcore_barrier()

    @pl.loop(0, cpt_max)
    def _(j):
        @pl.when(j < cnt)
        def _():
            pltpu.sync_copy(yw_hbm.at[src_v.at[j]], msg_v)
            pltpu.sync_copy(msg_v, acc_sh.at[dst_v.at[j]], add=True)

    plsc.subcore_barrier()
    pltpu.sync_copy(
        acc_sh.at[pl.ds(s * ROWS_PT, ROWS_PT)],
        out_hbm.at[c, pl.ds(s * ROWS_PT, ROWS_PT)],
    )


def _make_scatter(D, cpt0):
    cpt_max = max(cpt0, 2 * CPT - cpt0)
    return pl.kernel(
        functools.partial(_scatter_kernel, D, cpt0),
        out_type=jax.ShapeDtypeStruct((2, N_PAD, D), jnp.float32),
        mesh=_mesh(),
        scratch_types=[
            pltpu.VMEM((cpt_max, CHUNK), jnp.int32),
            pltpu.VMEM((cpt_max, CHUNK), jnp.int32),
            pltpu.VMEM((CHUNK, D), jnp.float32),
            pltpu.VMEM_SHARED((N_PAD, D), jnp.float32),
        ],
    )


_BM = 2048


def _dinv_of(dp_ref):
    deg = dp_ref[0, :] + dp_ref[1, :] + 1.0
    return lax.rsqrt(deg)


def _k1_body(x_ref, w_ref, dp_ref, o_ref):
    dinv = _dinv_of(dp_ref)
    xw = jnp.dot(x_ref[...], w_ref[...], preferred_element_type=jnp.float32)
    o_ref[...] = xw * dinv[:, None]


def _k2_body(sp_ref, yw_ref, dp_ref, w_ref, b_ref, o_ref):
    dinv = _dinv_of(dp_ref)
    stot = sp_ref[0] + sp_ref[1] + yw_ref[...]
    h = jnp.maximum(stot * dinv[:, None] + b_ref[...], 0.0)
    o_ref[...] = jnp.dot(h, w_ref[...], preferred_element_type=jnp.float32) * dinv[:, None]


def _k3_body(sp_ref, yw_ref, dp_ref, b_ref, o_ref):
    dinv = _dinv_of(dp_ref)
    stot = sp_ref[0] + sp_ref[1] + yw_ref[...]
    o_ref[...] = (stot * dinv[:, None])[:, :D_OUT] + b_ref[...]


def _row_spec(d):
    return pl.BlockSpec((_BM, d), lambda i: (i, 0))


def _parts_spec(d):
    return pl.BlockSpec((2, _BM, d), lambda i: (0, i, 0))


_DP_SPEC = pl.BlockSpec((2, _BM), lambda i: (0, i))
_GRID = (N_PAD // _BM,)


def _k1(x_pad, W1, deg_parts):
    return pl.pallas_call(
        _k1_body,
        grid=_GRID,
        in_specs=[
            _row_spec(D_IN),
            pl.BlockSpec((D_IN, D_H), lambda i: (0, 0)),
            _DP_SPEC,
        ],
        out_specs=_row_spec(D_H),
        out_shape=jax.ShapeDtypeStruct((N_PAD, D_H), jnp.float32),
    )(x_pad, W1, deg_parts)


def _k2(s1_parts, yw1, deg_parts, W2, b1_2d):
    return pl.pallas_call(
        _k2_body,
        grid=_GRID,
        in_specs=[
            _parts_spec(D_H),
            _row_spec(D_H),
            _DP_SPEC,
            pl.BlockSpec((D_H, D_H), lambda i: (0, 0)),
            pl.BlockSpec((1, D_H), lambda i: (0, 0)),
        ],
        out_specs=_row_spec(D_H),
        out_shape=jax.ShapeDtypeStruct((N_PAD, D_H), jnp.float32),
    )(s1_parts, yw1, deg_parts, W2, b1_2d)


def _k3(s2_parts, yw2, deg_parts, b2_2d):
    return pl.pallas_call(
        _k3_body,
        grid=_GRID,
        in_specs=[
            _parts_spec(D_H),
            _row_spec(D_H),
            _DP_SPEC,
            pl.BlockSpec((1, D_OUT), lambda i: (0, 0)),
        ],
        out_specs=_row_spec(D_OUT),
        out_shape=jax.ShapeDtypeStruct((N_PAD, D_OUT), jnp.float32),
    )(s2_parts, yw2, deg_parts, b2_2d)


@jax.jit
def _run(x, edge_index, W1, b1, W2, b2):
    x_pad = jnp.pad(x, ((0, N_PAD - N), (0, 0)))
    ei = edge_index.astype(jnp.int32)
    pad_idx = jnp.full((E_PAD - E,), N_PAD - 1, jnp.int32)
    src2d = jnp.concatenate([ei[0], pad_idx]).reshape(E2D_ROWS, CHUNK)
    dst2d = jnp.concatenate([ei[1], pad_idx]).reshape(E2D_ROWS, CHUNK)

    deg_parts = _make_deg()(dst2d)
    yw1 = _k1(x_pad, W1, deg_parts)
    s1_parts = _make_scatter(D_H, 128)(yw1, src2d, dst2d)
    W2p = jnp.pad(W2, ((0, 0), (0, D_H - D_OUT)))
    yw2 = _k2(s1_parts, yw1, deg_parts, W2p, b1.reshape(1, D_H))
    s2_parts = _make_scatter(D_H, 128)(yw2, src2d, dst2d)
    out = _k3(s2_parts, yw2, deg_parts, b2.reshape(1, D_OUT))
    return out[:N]


def kernel(x, edge_index, W1, b1, W2, b2):
    return _run(x, edge_index, W1, b1, W2, b2)
